# baseline (speedup 1.0000x reference)
import functools

import jax
import jax.numpy as jnp
from jax import lax
from jax.experimental import pallas as pl
from jax.experimental.pallas import tpu as pltpu
from jax.experimental.pallas import tpu_sc as plsc

_NC = 2
_NS = 16
_NT = _NC * _NS
_L = 16
_W = 64
_CH = 64
_NB = 4
_PF = 2


def _splat_lane0(v):
    lane0 = lax.iota(jnp.int32, _L) == 0
    return jnp.sum(jnp.where(lane0, v, 0))


@functools.partial(jax.jit, static_argnames=("n_rows", "n_upd"))
def _scatter_overwrite(x, idx, upd, *, n_rows, n_upd):
    rows_main = (-(-n_rows // _NT) + 15) // 16 * 16
    rows_last = n_rows - (_NT - 1) * rows_main
    assert 0 < rows_last <= rows_main and n_rows % 8 == 0
    assert rows_main % _CH == 0
    n_full_last = rows_last // _CH
    tail_last = rows_last - n_full_last * _CH
    assert tail_last % 8 == 0
    row_w = x.shape[1]
    n_chunks = n_upd // _L
    n_rchunks = rows_main // _L

    mesh = plsc.VectorSubcoreMesh(core_axis_name="c", subcore_axis_name="s")

    @functools.partial(
        pl.kernel,
        mesh=mesh,
        out_type=jax.ShapeDtypeStruct((n_rows, row_w), jnp.float32),
        compiler_params=pltpu.CompilerParams(needs_layout_passes=False),
        scratch_types=[
            pltpu.VMEM((n_upd,), jnp.int32),
            pltpu.VMEM((rows_main,), jnp.int32),
            pltpu.VMEM((_W + rows_main,), jnp.int32),
            pltpu.VMEM((_W + rows_main,), jnp.int32),
            pltpu.VMEM((_W,), jnp.int32),
            pltpu.VMEM((_W,), jnp.int32),
            pltpu.VMEM((_W, row_w), jnp.float32),
            [pltpu.VMEM((_CH, row_w), jnp.float32) for _ in range(_NB)],
            [pltpu.SemaphoreType.DMA for _ in range(_NB)],
            [pltpu.SemaphoreType.DMA for _ in range(_NB)],
            pltpu.SemaphoreType.DMA,
            pltpu.SemaphoreType.DMA,
        ],
    )
    def run(x_hbm, idx_hbm, upd_hbm, out_hbm,
            idx_v, claim, tgt_list, pos_list, tgt_buf, pos_buf, upd_stage,
            cbufs, isems, osems, gsem, ssem):
        wid = lax.axis_index("s") * _NC + lax.axis_index("c")
        lo = wid * rows_main
        hi = jnp.where(wid == _NT - 1, jnp.int32(n_rows), lo + rows_main)

        pltpu.sync_copy(idx_hbm, idx_v)

        neg1 = jnp.full((_L,), -1, jnp.int32)

        def init(k, _):
            claim[pl.ds(k * _L, _L)] = neg1
            return 0

        lax.fori_loop(0, n_rchunks, init, 0)

        def p1(c1):
            v = idx_v[pl.ds(c1 * _L, _L)]
            m = (v >= lo) & (v < hi)
            local = v - lo
            p = c1 * _L + lax.iota(jnp.int32, _L)
            _, lastm = plsc.scan_count(local, mask=m)
            plsc.store_scatter(claim, [local], p, mask=lastm)

        def in_dma(c, u):
            return pltpu.make_async_copy(
                x_hbm.at[pl.ds(lo + c * _CH, _CH)], cbufs[u], isems[u])

        def out_dma(c, u):
            return pltpu.make_async_copy(
                cbufs[u], out_hbm.at[pl.ds(lo + c * _CH, _CH)], osems[u])

        def copy_pipeline(n_cc):
            n_outer = -(-n_cc // _NB)
            f1_per = n_chunks // n_outer
            for u in range(min(_PF, n_cc)):
                in_dma(u, u).start()

            def body(i, _):
                for u in range(_NB):
                    c = i * _NB + u
                    pf = c + _PF
                    ub = (u + _PF) % _NB

                    @pl.when((pf < n_cc) & (pf >= _NB))
                    def _():
                        out_dma(pf - _NB, ub).wait()
                        in_dma(pf, ub).start()

                    @pl.when((pf < n_cc) & (pf < _NB))
                    def _():
                        in_dma(pf, ub).start()

                    @pl.when(c < n_cc)
                    def _():
                        in_dma(c, u).wait()
                        out_dma(c, u).start()
                for j in range(f1_per):
                    p1(i * f1_per + j)
                return 0

            lax.fori_loop(0, n_outer, body, 0)
            for c1 in range(n_outer * f1_per, n_chunks):
                p1(c1)
            for v in range(min(_NB, n_cc)):
                out_dma(n_cc - min(_NB, n_cc) + v,
                        (n_cc - min(_NB, n_cc) + v) % _NB).wait()

        @pl.when(wid < _NT - 1)
        def _():
            copy_pipeline(rows_main // _CH)

        @pl.when(wid == _NT - 1)
        def _():
            copy_pipeline(n_full_last)
            if tail_last:
                t0 = lo + n_full_last * _CH
                d = pltpu.make_async_copy(
                    x_hbm.at[pl.ds(t0, tail_last)],
                    cbufs[0].at[pl.ds(0, tail_last)], isems[0])
                d.start()
                d.wait()
                d = pltpu.make_async_copy(
                    cbufs[0].at[pl.ds(0, tail_last)],
                    out_hbm.at[pl.ds(t0, tail_last)], osems[0])
                d.start()
                d.wait()

        def emit(k, off):
            c = claim[pl.ds(k * _L, _L)]
            m = c >= 0
            cnt = jnp.sum(m.astype(jnp.int32))
            tgt = lo + k * _L + lax.iota(jnp.int32, _L)
            plsc.store_compressed(tgt_list.at[pl.ds(off, _L)], tgt, mask=m)
            plsc.store_compressed(pos_list.at[pl.ds(off, _L)], c, mask=m)
            return off + cnt

        end = lax.fori_loop(0, n_rchunks, emit, jnp.int32(_W))
        count = end - _W

        @pl.when(count > 0)
        def _():
            ft = _splat_lane0(tgt_list[pl.ds(_W, _L)])
            fp = _splat_lane0(pos_list[pl.ds(_W, _L)])
            for k in range(_W // _L):
                tgt_list[pl.ds(k * _L, _L)] = jnp.full((_L,), ft, jnp.int32)
                pos_list[pl.ds(k * _L, _L)] = jnp.full((_L,), fp, jnp.int32)

            nwin = (count + _W - 1) // _W

            def win(j, _):
                st = end - (nwin - j) * _W
                for k in range(_W // _L):
                    tgt_buf[pl.ds(k * _L, _L)] = tgt_list[pl.ds(st + k * _L, _L)]
                    pos_buf[pl.ds(k * _L, _L)] = pos_list[pl.ds(st + k * _L, _L)]
                g = pltpu.make_async_copy(
                    upd_hbm.at[pos_buf], upd_stage, gsem)
                g.start()
                g.wait()
                s = pltpu.make_async_copy(
                    upd_stage, out_hbm.at[tgt_buf], ssem)
                s.start()
                s.wait()
                return 0

            lax.fori_loop(0, nwin, win, 0)

    return run(x, idx, upd)


def kernel(input, indices, update):
    n_rows = input.shape[0]
    n_upd = indices.shape[0]
    row_w = input.shape[1] * input.shape[2]
    x = input.reshape(n_rows, row_w)
    u = update.reshape(n_upd, row_w)
    idx = indices.astype(jnp.int32)
    out = _scatter_overwrite(x, idx, u, n_rows=n_rows, n_upd=n_upd)
    return out.reshape(input.shape)


# speedup vs baseline: 1.0103x; 1.0103x over previous
import functools

import jax
import jax.numpy as jnp
from jax import lax
from jax.experimental import pallas as pl
from jax.experimental.pallas import tpu as pltpu
from jax.experimental.pallas import tpu_sc as plsc

_NC = 2
_NS = 16
_NT = _NC * _NS
_L = 16
_W = 64
_CH = 128
_NB = 2


def _splat_lane0(v):
    lane0 = lax.iota(jnp.int32, _L) == 0
    return jnp.sum(jnp.where(lane0, v, 0))


@functools.partial(jax.jit, static_argnames=("n_rows", "n_upd"))
def _scatter_overwrite(x, idx, upd, *, n_rows, n_upd):
    rows_main = (-(-n_rows // _NT) + 15) // 16 * 16
    rows_last = n_rows - (_NT - 1) * rows_main
    assert 0 < rows_last <= rows_main and n_rows % 8 == 0
    n_full_main = rows_main // _CH
    tail_main = rows_main - n_full_main * _CH
    n_full_last = rows_last // _CH
    tail_last = rows_last - n_full_last * _CH
    assert tail_main % 8 == 0 and tail_last % 8 == 0
    row_w = x.shape[1]
    n_chunks = n_upd // _L
    n_rchunks = rows_main // _L

    mesh = plsc.VectorSubcoreMesh(core_axis_name="c", subcore_axis_name="s")

    @functools.partial(
        pl.kernel,
        mesh=mesh,
        out_type=jax.ShapeDtypeStruct((n_rows, row_w), jnp.float32),
        compiler_params=pltpu.CompilerParams(needs_layout_passes=False),
        scratch_types=[
            pltpu.VMEM((n_upd,), jnp.int32),
            pltpu.VMEM((rows_main,), jnp.int32),
            pltpu.VMEM((_W + rows_main,), jnp.int32),
            pltpu.VMEM((_W + rows_main,), jnp.int32),
            pltpu.VMEM((_W,), jnp.int32),
            pltpu.VMEM((_W,), jnp.int32),
            pltpu.VMEM((_W, row_w), jnp.float32),
            pltpu.VMEM_SHARED((_NS, _NB, _CH, row_w), jnp.float32),
            [pltpu.SemaphoreType.DMA for _ in range(_NB)],
            [pltpu.SemaphoreType.DMA for _ in range(_NB)],
            pltpu.SemaphoreType.DMA,
            pltpu.SemaphoreType.DMA,
        ],
    )
    def run(x_hbm, idx_hbm, upd_hbm, out_hbm,
            idx_v, claim, tgt_list, pos_list, tgt_buf, pos_buf, upd_stage,
            spbuf, isems, osems, gsem, ssem):
        wid = lax.axis_index("s") * _NC + lax.axis_index("c")
        lo = wid * rows_main
        hi = jnp.where(wid == _NT - 1, jnp.int32(n_rows), lo + rows_main)

        pltpu.sync_copy(idx_hbm, idx_v)

        neg1 = jnp.full((_L,), -1, jnp.int32)

        def init(k, _):
            claim[pl.ds(k * _L, _L)] = neg1
            return 0

        lax.fori_loop(0, n_rchunks, init, 0)

        def p1(c1):
            v = idx_v[pl.ds(c1 * _L, _L)]
            m = (v >= lo) & (v < hi)
            local = v - lo
            p = c1 * _L + lax.iota(jnp.int32, _L)
            _, lastm = plsc.scan_count(local, mask=m)
            plsc.store_scatter(claim, [local], p, mask=lastm)

        sid = lax.axis_index("s")

        def in_dma(c, u):
            return pltpu.make_async_copy(
                x_hbm.at[pl.ds(lo + c * _CH, _CH)], spbuf.at[sid, u],
                isems[u])

        def out_dma(c, u):
            return pltpu.make_async_copy(
                spbuf.at[sid, u], out_hbm.at[pl.ds(lo + c * _CH, _CH)],
                osems[u])

        def copy_pipeline(n_cc, tail):
            f1_per = n_chunks // n_cc

            def body(i, _):
                for u in range(_NB):
                    @pl.when(i * _NB + u >= _NB)
                    def _():
                        out_dma(i * _NB + u - _NB, u).wait()
                    d = in_dma(i * _NB + u, u)
                    d.start()
                    for j in range(f1_per):
                        p1((i * _NB + u) * f1_per + j)
                    d.wait()
                    out_dma(i * _NB + u, u).start()
                return 0

            lax.fori_loop(0, n_cc // _NB, body, 0)
            if n_cc % _NB:
                c = n_cc - 1
                u = c % _NB

                @pl.when(c >= _NB)
                def _():
                    out_dma(c - _NB, u).wait()
                d = in_dma(c, u)
                d.start()
                for j in range(f1_per):
                    p1(c * f1_per + j)
                d.wait()
                out_dma(c, u).start()
            for c1 in range(n_cc * f1_per, n_chunks):
                p1(c1)
            if tail:
                t0 = lo + n_cc * _CH
                tu = n_cc % _NB
                @pl.when(n_cc >= _NB)
                def _():
                    out_dma(n_cc - _NB, tu).wait()
                d = pltpu.make_async_copy(
                    x_hbm.at[pl.ds(t0, tail)],
                    spbuf.at[sid, tu, pl.ds(0, tail)], isems[tu])
                d.start()
                d.wait()
                d = pltpu.make_async_copy(
                    spbuf.at[sid, tu, pl.ds(0, tail)],
                    out_hbm.at[pl.ds(t0, tail)], osems[tu])
                d.start()
                d.wait()
            start_c = max(0, n_cc - _NB + (1 if (tail and n_cc >= _NB) else 0))
            for c in range(start_c, n_cc):
                out_dma(c, c % _NB).wait()

        @pl.when(wid < _NT - 1)
        def _():
            copy_pipeline(n_full_main, tail_main)

        @pl.when(wid == _NT - 1)
        def _():
            copy_pipeline(n_full_last, tail_last)

        def emit(k, off):
            c = claim[pl.ds(k * _L, _L)]
            m = c >= 0
            cnt = jnp.sum(m.astype(jnp.int32))
            tgt = lo + k * _L + lax.iota(jnp.int32, _L)
            plsc.store_compressed(tgt_list.at[pl.ds(off, _L)], tgt, mask=m)
            plsc.store_compressed(pos_list.at[pl.ds(off, _L)], c, mask=m)
            return off + cnt

        end = lax.fori_loop(0, n_rchunks, emit, jnp.int32(_W))
        count = end - _W

        @pl.when(count > 0)
        def _():
            ft = _splat_lane0(tgt_list[pl.ds(_W, _L)])
            fp = _splat_lane0(pos_list[pl.ds(_W, _L)])
            for k in range(_W // _L):
                tgt_list[pl.ds(k * _L, _L)] = jnp.full((_L,), ft, jnp.int32)
                pos_list[pl.ds(k * _L, _L)] = jnp.full((_L,), fp, jnp.int32)

            nwin = (count + _W - 1) // _W

            def win(j, _):
                st = end - (nwin - j) * _W
                for k in range(_W // _L):
                    tgt_buf[pl.ds(k * _L, _L)] = tgt_list[pl.ds(st + k * _L, _L)]
                    pos_buf[pl.ds(k * _L, _L)] = pos_list[pl.ds(st + k * _L, _L)]
                g = pltpu.make_async_copy(
                    upd_hbm.at[pos_buf], upd_stage, gsem)
                g.start()
                g.wait()
                s = pltpu.make_async_copy(
                    upd_stage, out_hbm.at[tgt_buf], ssem)
                s.start()
                s.wait()
                return 0

            lax.fori_loop(0, nwin, win, 0)

    return run(x, idx, upd)


def kernel(input, indices, update):
    n_rows = input.shape[0]
    n_upd = indices.shape[0]
    row_w = input.shape[1] * input.shape[2]
    x = input.reshape(n_rows, row_w)
    u = update.reshape(n_upd, row_w)
    idx = indices.astype(jnp.int32)
    out = _scatter_overwrite(x, idx, u, n_rows=n_rows, n_upd=n_upd)
    return out.reshape(input.shape)


# speedup vs baseline: 1.2212x; 1.2087x over previous
import functools

import jax
import jax.numpy as jnp
from jax import lax
from jax.experimental import pallas as pl
from jax.experimental.pallas import tpu as pltpu
from jax.experimental.pallas import tpu_sc as plsc

_NC = 2
_NS = 16
_NT = _NC * _NS
_L = 16
_W = 64


def _splat_lane0(v):
    lane0 = lax.iota(jnp.int32, _L) == 0
    return jnp.sum(jnp.where(lane0, v, 0))


@functools.partial(jax.jit, static_argnames=("n_rows", "n_upd"))
def _scatter_overwrite(x, idx, upd, *, n_rows, n_upd):
    rows_main = (-(-n_rows // _NT) + 15) // 16 * 16
    rows_last = n_rows - (_NT - 1) * rows_main
    assert 0 < rows_last <= rows_main and n_rows % 8 == 0
    row_w = x.shape[1]
    n_chunks = n_upd // _L
    n_rchunks = rows_main // _L

    mesh = plsc.VectorSubcoreMesh(core_axis_name="c", subcore_axis_name="s")

    def stateful(refs):
        x_ref, idx_ref, upd_ref = refs

        @pl.core_map(
            mesh,
            compiler_params=pltpu.CompilerParams(needs_layout_passes=False),
            scratch_shapes=[
                pltpu.VMEM((n_upd,), jnp.int32),
                pltpu.VMEM((rows_main,), jnp.int32),
                pltpu.VMEM((_W + rows_main,), jnp.int32),
                pltpu.VMEM((_W + rows_main,), jnp.int32),
                pltpu.VMEM((_W,), jnp.int32),
                pltpu.VMEM((_W,), jnp.int32),
                pltpu.VMEM((_W, row_w), jnp.float32),
                pltpu.SemaphoreType.DMA,
                pltpu.SemaphoreType.DMA,
            ],
        )
        def _(idx_v, claim, tgt_list, pos_list, tgt_buf, pos_buf, upd_stage,
              gsem, ssem):
            wid = lax.axis_index("s") * _NC + lax.axis_index("c")
            lo = wid * rows_main
            hi = jnp.where(wid == _NT - 1, jnp.int32(n_rows), lo + rows_main)

            pltpu.sync_copy(idx_ref, idx_v)

            neg1 = jnp.full((_L,), -1, jnp.int32)

            def init(k, _):
                claim[pl.ds(k * _L, _L)] = neg1
                return 0

            lax.fori_loop(0, n_rchunks, init, 0)

            def p1(i, _):
                v = idx_v[pl.ds(i * _L, _L)]
                m = (v >= lo) & (v < hi)
                local = v - lo
                p = i * _L + lax.iota(jnp.int32, _L)
                _, lastm = plsc.scan_count(local, mask=m)
                plsc.store_scatter(claim, [local], p, mask=lastm)
                return 0

            lax.fori_loop(0, n_chunks, p1, 0)

            def emit(k, off):
                c = claim[pl.ds(k * _L, _L)]
                m = c >= 0
                cnt = jnp.sum(m.astype(jnp.int32))
                tgt = lo + k * _L + lax.iota(jnp.int32, _L)
                plsc.store_compressed(tgt_list.at[pl.ds(off, _L)], tgt,
                                      mask=m)
                plsc.store_compressed(pos_list.at[pl.ds(off, _L)], c, mask=m)
                return off + cnt

            end = lax.fori_loop(0, n_rchunks, emit, jnp.int32(_W))
            count = end - _W

            @pl.when(count > 0)
            def _():
                ft = _splat_lane0(tgt_list[pl.ds(_W, _L)])
                fp = _splat_lane0(pos_list[pl.ds(_W, _L)])
                for k in range(_W // _L):
                    tgt_list[pl.ds(k * _L, _L)] = jnp.full((_L,), ft,
                                                           jnp.int32)
                    pos_list[pl.ds(k * _L, _L)] = jnp.full((_L,), fp,
                                                           jnp.int32)

                nwin = (count + _W - 1) // _W

                def win(j, _):
                    st = end - (nwin - j) * _W
                    for k in range(_W // _L):
                        tgt_buf[pl.ds(k * _L, _L)] = (
                            tgt_list[pl.ds(st + k * _L, _L)])
                        pos_buf[pl.ds(k * _L, _L)] = (
                            pos_list[pl.ds(st + k * _L, _L)])
                    g = pltpu.make_async_copy(
                        upd_ref.at[pos_buf], upd_stage, gsem)
                    g.start()
                    g.wait()
                    s = pltpu.make_async_copy(
                        upd_stage, x_ref.at[tgt_buf], ssem)
                    s.start()
                    s.wait()
                    return 0

                lax.fori_loop(0, nwin, win, 0)

    x_fin, _, _ = pl.run_state(stateful)((x, idx, upd))
    return x_fin


def kernel(input, indices, update):
    n_rows = input.shape[0]
    n_upd = indices.shape[0]
    row_w = input.shape[1] * input.shape[2]
    x = input.reshape(n_rows, row_w)
    u = update.reshape(n_upd, row_w)
    idx = indices.astype(jnp.int32)
    out = _scatter_overwrite(x, idx, u, n_rows=n_rows, n_upd=n_upd)
    return out.reshape(input.shape)
